# baseline (speedup 1.0000x reference)
import functools

import jax
import jax.numpy as jnp
from jax import lax
from jax.experimental import pallas as pl
from jax.experimental.pallas import tpu as pltpu
from jax.experimental.pallas import tpu_sc as plsc

TT = 256
TILE = 128
LANES = 128


def _router_body(x_ref, wt_ref, b_ref, eid_ref, rank_ref, hist_ref):
    i = pl.program_id(0)

    @pl.when(i == 0)
    def _init():
        hist_ref[...] = jnp.zeros_like(hist_ref)

    x = x_ref[...]
    logits = jnp.dot(x, wt_ref[...], preferred_element_type=jnp.float32)
    logits = logits + b_ref[0:1, :]
    lane = lax.broadcasted_iota(jnp.int32, logits.shape, 1)
    m = jnp.max(logits, axis=1, keepdims=True)
    cand = jnp.where(logits >= m, lane, LANES)
    eid = jnp.min(cand, axis=1, keepdims=True)
    onehot = (lane == eid).astype(jnp.float32)

    r = lax.broadcasted_iota(jnp.int32, (TT, TT), 0)
    c = lax.broadcasted_iota(jnp.int32, (TT, TT), 1)
    lt = (c < r).astype(jnp.float32)
    prefix = jnp.dot(lt, onehot, preferred_element_type=jnp.float32)
    run = hist_ref[0:1, :]
    rank = (jnp.sum(prefix * onehot, axis=1, keepdims=True)
            + jnp.sum(onehot * run, axis=1, keepdims=True))

    eid_ref[...] = eid
    rank_ref[...] = rank.astype(jnp.int32)
    hist_ref[...] = hist_ref[...] + jnp.sum(onehot, axis=0, keepdims=True)


def _route(tokens, wt_pad, b_pad):
    t, h = tokens.shape
    nt = t // TT
    eid, rank, hist = pl.pallas_call(
        _router_body,
        grid=(nt,),
        in_specs=[
            pl.BlockSpec((TT, h), lambda i: (i, 0)),
            pl.BlockSpec((h, LANES), lambda i: (0, 0)),
            pl.BlockSpec((8, LANES), lambda i: (0, 0)),
        ],
        out_specs=[
            pl.BlockSpec((TT, 1), lambda i: (i, 0)),
            pl.BlockSpec((TT, 1), lambda i: (i, 0)),
            pl.BlockSpec((8, LANES), lambda i: (0, 0)),
        ],
        out_shape=[
            jax.ShapeDtypeStruct((t, 1), jnp.int32),
            jax.ShapeDtypeStruct((t, 1), jnp.int32),
            jax.ShapeDtypeStruct((8, LANES), jnp.float32),
        ],
    )(tokens, wt_pad, b_pad)
    return eid.reshape(t), rank.reshape(t), hist[0, :].astype(jnp.int32)


def _pos_body(eid_ref, rank_ref, base_ref, pos_ref):
    lane = lax.broadcasted_iota(jnp.int32, (eid_ref.shape[0], LANES), 1)
    onehot = (lane == eid_ref[...]).astype(jnp.float32)
    basef = base_ref[0:1, :].astype(jnp.float32)
    off = jnp.sum(onehot * basef, axis=1, keepdims=True)
    pos_ref[...] = rank_ref[...] + off.astype(jnp.int32)


def _posadd(eid, rank, base_pad):
    t = eid.shape[0]
    nt = t // TT
    pos = pl.pallas_call(
        _pos_body,
        grid=(nt,),
        in_specs=[
            pl.BlockSpec((TT, 1), lambda i: (i, 0)),
            pl.BlockSpec((TT, 1), lambda i: (i, 0)),
            pl.BlockSpec((8, LANES), lambda i: (0, 0)),
        ],
        out_specs=pl.BlockSpec((TT, 1), lambda i: (i, 0)),
        out_shape=jax.ShapeDtypeStruct((t, 1), jnp.int32),
    )(eid.reshape(t, 1), rank.reshape(t, 1), base_pad)
    return pos.reshape(t)


def _dispatch(tokens, pos, n_sorted):
    t, h = tokens.shape
    info = plsc.get_sparse_core_info()
    nw = info.num_cores * info.num_subcores
    cpw = t // nw
    mesh = plsc.VectorSubcoreMesh(core_axis_name="c", subcore_axis_name="s")

    @functools.partial(
        pl.kernel,
        mesh=mesh,
        out_type=jax.ShapeDtypeStruct((n_sorted, h), jnp.float32),
        scratch_types=[
            pltpu.VMEM((cpw,), jnp.int32),
            pltpu.VMEM((cpw, h), jnp.float32),
            pltpu.SemaphoreType.DMA,
        ],
    )
    def k(tokens_hbm, pos_hbm, xs_hbm, pos_v, rows_v, sem):
        wid = lax.axis_index("s") * info.num_cores + lax.axis_index("c")
        off = wid * cpw
        pltpu.sync_copy(pos_hbm.at[pl.ds(off, cpw)], pos_v)
        pltpu.sync_copy(tokens_hbm.at[pl.ds(off, cpw)], rows_v)
        pltpu.async_copy(rows_v, xs_hbm.at[pos_v], sem).wait()

    return k(tokens, pos)


def _ffn_body(s_ref, x_ref, w1_ref, w2_ref, y_ref, w1c_ref, w2c_ref, *,
              ntiles):
    i = pl.program_id(0)

    @pl.when(jnp.logical_or(i == 0, s_ref[i] != s_ref[jnp.maximum(i - 1, 0)]))
    def _cast():
        w1c_ref[...] = w1_ref[0].astype(jnp.bfloat16)
        w2c_ref[...] = w2_ref[0].astype(jnp.bfloat16)

    @pl.when(i < s_ref[ntiles])
    def _():
        x = x_ref[...].astype(jnp.bfloat16)
        hmid = lax.dot_general(x, w1c_ref[...], (((1,), (1,)), ((), ())),
                               preferred_element_type=jnp.float32)
        hmid = hmid * jax.nn.sigmoid(hmid)
        y_ref[...] = lax.dot_general(hmid.astype(jnp.bfloat16), w2c_ref[...],
                                     (((1,), (1,)), ((), ())),
                                     preferred_element_type=jnp.float32)


def _ffn(sinfo, xs, w1, w2):
    ns, h = xs.shape
    e, f, _ = w1.shape
    ntiles = ns // TILE
    grid_spec = pltpu.PrefetchScalarGridSpec(
        num_scalar_prefetch=1,
        grid=(ntiles,),
        in_specs=[
            pl.BlockSpec((TILE, h), lambda i, s: (i, 0)),
            pl.BlockSpec((1, f, h), lambda i, s: (s[i], 0, 0)),
            pl.BlockSpec((1, h, f), lambda i, s: (s[i], 0, 0)),
        ],
        out_specs=pl.BlockSpec((TILE, h), lambda i, s: (i, 0)),
        scratch_shapes=[
            pltpu.VMEM((f, h), jnp.bfloat16),
            pltpu.VMEM((h, f), jnp.bfloat16),
        ],
    )
    return pl.pallas_call(
        functools.partial(_ffn_body, ntiles=ntiles),
        grid_spec=grid_spec,
        out_shape=jax.ShapeDtypeStruct((ns, h), jnp.float32),
    )(sinfo, xs, w1, w2)


def _combine(ys, pos):
    ns, h = ys.shape
    t = pos.shape[0]
    info = plsc.get_sparse_core_info()
    nw = info.num_cores * info.num_subcores
    cpw = t // nw
    mesh = plsc.VectorSubcoreMesh(core_axis_name="c", subcore_axis_name="s")

    @functools.partial(
        pl.kernel,
        mesh=mesh,
        out_type=jax.ShapeDtypeStruct((t, h), jnp.float32),
        scratch_types=[
            pltpu.VMEM((cpw,), jnp.int32),
            pltpu.VMEM((cpw, h), jnp.float32),
            pltpu.SemaphoreType.DMA,
        ],
    )
    def k(ys_hbm, pos_hbm, out_hbm, idx_v, rows_v, sem):
        wid = lax.axis_index("s") * info.num_cores + lax.axis_index("c")
        off = wid * cpw
        pltpu.sync_copy(pos_hbm.at[pl.ds(off, cpw)], idx_v)
        pltpu.async_copy(ys_hbm.at[idx_v], rows_v, sem).wait()
        pltpu.sync_copy(rows_v, out_hbm.at[pl.ds(off, cpw)])

    return k(ys, pos)


def kernel(hidden_states, router_w, router_b, w1, w2):
    b, s, h = hidden_states.shape
    e, f, _ = w1.shape
    tokens = hidden_states.reshape(-1, h)
    t = tokens.shape[0]

    wt_pad = jnp.zeros((h, LANES), jnp.float32).at[:, :e].set(router_w.T)
    b_row = jnp.full((LANES,), -1e30, jnp.float32).at[:e].set(router_b)
    b_pad = jnp.broadcast_to(b_row, (8, LANES))

    eid, rank, hist = _route(tokens, wt_pad, b_pad)
    counts = hist[:e]

    padded = ((counts + TILE - 1) // TILE) * TILE
    base = jnp.concatenate([jnp.zeros((1,), jnp.int32),
                            jnp.cumsum(padded)[:-1].astype(jnp.int32)])
    num_used = jnp.sum(padded) // TILE
    n_sorted = t + e * TILE
    ntiles = n_sorted // TILE
    starts = jnp.arange(ntiles, dtype=jnp.int32) * TILE
    teid_raw = jnp.sum((starts[:, None] >= base[None, :]).astype(jnp.int32),
                       axis=1) - 1
    last_eid = jnp.take(teid_raw, num_used - 1)
    teids = jnp.where(jnp.arange(ntiles) < num_used, teid_raw, last_eid)
    sinfo = jnp.concatenate([teids, num_used[None]]).astype(jnp.int32)
    base_pad = jnp.broadcast_to(
        jnp.zeros((LANES,), jnp.int32).at[:e].set(base), (8, LANES))

    pos = _posadd(eid, rank, base_pad)
    xs = _dispatch(tokens, pos, n_sorted)
    ys = _ffn(sinfo, xs, w1, w2)
    out = _combine(ys, pos)
    return out.reshape(b, s, h)


# speedup vs baseline: 1.0446x; 1.0446x over previous
import functools

import jax
import jax.numpy as jnp
from jax import lax
from jax.experimental import pallas as pl
from jax.experimental.pallas import tpu as pltpu
from jax.experimental.pallas import tpu_sc as plsc

TT = 256
TILE = 128
LANES = 128


def _router_body(x_ref, wt_ref, b_ref, eid_ref, rank_ref, hist_ref):
    i = pl.program_id(0)

    @pl.when(i == 0)
    def _init():
        hist_ref[...] = jnp.zeros_like(hist_ref)

    x = x_ref[...]
    logits = jnp.dot(x, wt_ref[...], preferred_element_type=jnp.float32)
    logits = logits + b_ref[0:1, :]
    lane = lax.broadcasted_iota(jnp.int32, logits.shape, 1)
    m = jnp.max(logits, axis=1, keepdims=True)
    cand = jnp.where(logits >= m, lane, LANES)
    eid = jnp.min(cand, axis=1, keepdims=True)
    onehot = (lane == eid).astype(jnp.float32)

    r = lax.broadcasted_iota(jnp.int32, (TT, TT), 0)
    c = lax.broadcasted_iota(jnp.int32, (TT, TT), 1)
    lt = (c < r).astype(jnp.float32)
    prefix = jnp.dot(lt, onehot, preferred_element_type=jnp.float32)
    run = hist_ref[0:1, :]
    rank = (jnp.sum(prefix * onehot, axis=1, keepdims=True)
            + jnp.sum(onehot * run, axis=1, keepdims=True))

    eid_ref[...] = eid
    rank_ref[...] = rank.astype(jnp.int32)
    hist_ref[...] = hist_ref[...] + jnp.sum(onehot, axis=0, keepdims=True)


def _route(tokens, wt_pad, b_pad):
    t, h = tokens.shape
    nt = t // TT
    eid, rank, hist = pl.pallas_call(
        _router_body,
        grid=(nt,),
        in_specs=[
            pl.BlockSpec((TT, h), lambda i: (i, 0)),
            pl.BlockSpec((h, LANES), lambda i: (0, 0)),
            pl.BlockSpec((8, LANES), lambda i: (0, 0)),
        ],
        out_specs=[
            pl.BlockSpec((TT, 1), lambda i: (i, 0)),
            pl.BlockSpec((TT, 1), lambda i: (i, 0)),
            pl.BlockSpec((8, LANES), lambda i: (0, 0)),
        ],
        out_shape=[
            jax.ShapeDtypeStruct((t, 1), jnp.int32),
            jax.ShapeDtypeStruct((t, 1), jnp.int32),
            jax.ShapeDtypeStruct((8, LANES), jnp.float32),
        ],
    )(tokens, wt_pad, b_pad)
    return eid, rank, hist


def _pos_body(eid_ref, rank_ref, hist_ref, pos_ref, meta_ref, *, ntiles):
    i = pl.program_id(0)
    nrow = eid_ref.shape[0]
    countf = hist_ref[0:1, :]
    padded = (((countf.astype(jnp.int32) + (TILE - 1)) >> 7) << 7)
    paddedf = padded.astype(jnp.float32)
    r = lax.broadcasted_iota(jnp.int32, (LANES, LANES), 0)
    c = lax.broadcasted_iota(jnp.int32, (LANES, LANES), 1)
    ltf = (r < c).astype(jnp.float32)
    paddedb = jnp.broadcast_to(paddedf, (8, LANES))
    basef = jnp.dot(paddedb, ltf, preferred_element_type=jnp.float32)[0:1, :]

    lane = lax.broadcasted_iota(jnp.int32, (nrow, LANES), 1)
    onehot = (lane == eid_ref[...]).astype(jnp.float32)
    off = jnp.sum(onehot * basef, axis=1, keepdims=True)
    pos_ref[...] = rank_ref[...] + off.astype(jnp.int32)

    @pl.when(i == 0)
    def _meta():
        validf = (c[0:1, :] < 8).astype(jnp.float32)
        totalf = jnp.sum(paddedf * validf)
        nu = (totalf * (1.0 / TILE)).astype(jnp.int32)
        base_b = jnp.broadcast_to(basef, (LANES, LANES))
        startf = (r * TILE).astype(jnp.float32)
        valid_b = c < 8
        cnt = jnp.sum(jnp.where(jnp.logical_and(base_b <= startf, valid_b),
                                1, 0), axis=1, keepdims=True)
        last = jnp.sum(jnp.where(
            jnp.logical_and(basef <= totalf - TILE, valid_b[0:1, :]), 1, 0)) - 1
        teids = jnp.minimum(cnt - 1, last)
        rows = lax.broadcasted_iota(jnp.int32, (LANES, 1), 0)
        meta_ref[...] = jnp.where(rows == ntiles, nu, teids)


def _posadd(eid, rank, hist_pad, ntiles):
    t = eid.shape[0]
    nt = t // TT
    pos, meta = pl.pallas_call(
        functools.partial(_pos_body, ntiles=ntiles),
        grid=(nt,),
        in_specs=[
            pl.BlockSpec((TT, 1), lambda i: (i, 0)),
            pl.BlockSpec((TT, 1), lambda i: (i, 0)),
            pl.BlockSpec((8, LANES), lambda i: (0, 0)),
        ],
        out_specs=[
            pl.BlockSpec((TT, 1), lambda i: (i, 0)),
            pl.BlockSpec((LANES, 1), lambda i: (0, 0)),
        ],
        out_shape=[
            jax.ShapeDtypeStruct((t, 1), jnp.int32),
            jax.ShapeDtypeStruct((LANES, 1), jnp.int32),
        ],
    )(eid.reshape(t, 1), rank.reshape(t, 1), hist_pad)
    return pos.reshape(t), meta.reshape(LANES)


def _dispatch(tokens, pos, n_sorted):
    t, h = tokens.shape
    info = plsc.get_sparse_core_info()
    nw = info.num_cores * info.num_subcores
    cpw = t // nw
    mesh = plsc.VectorSubcoreMesh(core_axis_name="c", subcore_axis_name="s")

    @functools.partial(
        pl.kernel,
        mesh=mesh,
        out_type=jax.ShapeDtypeStruct((n_sorted, h), jnp.float32),
        scratch_types=[
            pltpu.VMEM((cpw,), jnp.int32),
            pltpu.VMEM((cpw, h), jnp.float32),
            pltpu.SemaphoreType.DMA,
        ],
    )
    def k(tokens_hbm, pos_hbm, xs_hbm, pos_v, rows_v, sem):
        wid = lax.axis_index("s") * info.num_cores + lax.axis_index("c")
        off = wid * cpw
        pltpu.sync_copy(pos_hbm.at[pl.ds(off, cpw)], pos_v)
        pltpu.sync_copy(tokens_hbm.at[pl.ds(off, cpw)], rows_v)
        pltpu.async_copy(rows_v, xs_hbm.at[pos_v], sem).wait()

    return k(tokens, pos)


def _ffn_body(s_ref, x_ref, w1_ref, w2_ref, y_ref, *, ntiles):
    i = pl.program_id(0)

    @pl.when(i < s_ref[ntiles])
    def _():
        x = x_ref[...]
        hmid = lax.dot_general(x, w1_ref[0], (((1,), (1,)), ((), ())),
                               preferred_element_type=jnp.float32)
        hmid = hmid * jax.nn.sigmoid(hmid)
        y_ref[...] = lax.dot_general(hmid, w2_ref[0], (((1,), (1,)), ((), ())),
                                     preferred_element_type=jnp.float32)


def _ffn(sinfo, xs, w1, w2):
    ns, h = xs.shape
    e, f, _ = w1.shape
    ntiles = ns // TILE
    grid_spec = pltpu.PrefetchScalarGridSpec(
        num_scalar_prefetch=1,
        grid=(ntiles,),
        in_specs=[
            pl.BlockSpec((TILE, h), lambda i, s: (i, 0)),
            pl.BlockSpec((1, f, h), lambda i, s: (s[i], 0, 0)),
            pl.BlockSpec((1, h, f), lambda i, s: (s[i], 0, 0)),
        ],
        out_specs=pl.BlockSpec((TILE, h), lambda i, s: (i, 0)),
    )
    return pl.pallas_call(
        functools.partial(_ffn_body, ntiles=ntiles),
        grid_spec=grid_spec,
        out_shape=jax.ShapeDtypeStruct((ns, h), jnp.float32),
    )(sinfo, xs, w1, w2)


def _combine(ys, pos):
    ns, h = ys.shape
    t = pos.shape[0]
    info = plsc.get_sparse_core_info()
    nw = info.num_cores * info.num_subcores
    cpw = t // nw
    mesh = plsc.VectorSubcoreMesh(core_axis_name="c", subcore_axis_name="s")

    @functools.partial(
        pl.kernel,
        mesh=mesh,
        out_type=jax.ShapeDtypeStruct((t, h), jnp.float32),
        scratch_types=[
            pltpu.VMEM((cpw,), jnp.int32),
            pltpu.VMEM((cpw, h), jnp.float32),
            pltpu.SemaphoreType.DMA,
        ],
    )
    def k(ys_hbm, pos_hbm, out_hbm, idx_v, rows_v, sem):
        wid = lax.axis_index("s") * info.num_cores + lax.axis_index("c")
        off = wid * cpw
        pltpu.sync_copy(pos_hbm.at[pl.ds(off, cpw)], idx_v)
        pltpu.async_copy(ys_hbm.at[idx_v], rows_v, sem).wait()
        pltpu.sync_copy(rows_v, out_hbm.at[pl.ds(off, cpw)])

    return k(ys, pos)


def kernel(hidden_states, router_w, router_b, w1, w2):
    b, s, h = hidden_states.shape
    e, f, _ = w1.shape
    tokens = hidden_states.reshape(-1, h)
    t = tokens.shape[0]

    wt_pad = jnp.zeros((h, LANES), jnp.float32).at[:, :e].set(router_w.T)
    b_row = jnp.full((LANES,), -1e30, jnp.float32).at[:e].set(router_b)
    b_pad = jnp.broadcast_to(b_row, (8, LANES))

    n_sorted = t + e * TILE
    ntiles = n_sorted // TILE

    eid, rank, hist = _route(tokens, wt_pad, b_pad)
    pos, sinfo = _posadd(eid, rank, hist, ntiles)
    xs = _dispatch(tokens, pos, n_sorted)
    ys = _ffn(sinfo, xs, w1, w2)
    out = _combine(ys, pos)
    return out.reshape(b, s, h)
